# baseline (speedup 1.0000x reference)
import jax
import jax.numpy as jnp
from jax.experimental import pallas as pl

_N_CLUSTERS = 1000


def _combine(sums, counts):
  def body(p_ref, c_ref, o_ref):
    o_ref[...] = p_ref[...] / jnp.maximum(c_ref[..., 0:1], 1.0)

  return pl.pallas_call(
      body,
      out_shape=jax.ShapeDtypeStruct(sums.shape, jnp.float32),
  )(sums, counts)


def kernel(X, assign):
  sums = jax.ops.segment_sum(X, assign, num_segments=_N_CLUSTERS)
  counts = jax.ops.segment_sum(
      jnp.ones((X.shape[0],), X.dtype), assign, num_segments=_N_CLUSTERS)
  counts = jnp.broadcast_to(counts[:, None], (_N_CLUSTERS, 128))
  return _combine(sums, counts)


# speedup vs baseline: 1.7361x; 1.7361x over previous
import functools

import jax
import jax.numpy as jnp
from jax import lax
from jax.experimental import pallas as pl
from jax.experimental.pallas import tpu as pltpu
from jax.experimental.pallas import tpu_sc as plsc

_N_ROWS = 100000
_D = 512
_NG = _D // 16
_N_CLUSTERS = 1000
_C_PAD = 1024
_NC = 2
_NS = 16
_NW = _NC * _NS
_CW = 128
_CHUNK = 80
_GPC = _CHUNK // 16
_N_CHUNKS = _N_ROWS // _CHUNK
_BASE = _N_CHUNKS // _NW
_EXTRA = _N_CHUNKS - _BASE * _NW
_SLAB = _C_PAD // _NS


def _tile_bounds(w):
  c0 = _BASE * w + min(w, _EXTRA)
  n = _BASE + (1 if w < _EXTRA else 0)
  return c0 * _CHUNK, (c0 + n) * _CHUNK


def _sc_partial_sums(x, a, zrow, zcnt):
  mesh = plsc.VectorSubcoreMesh(core_axis_name="c", subcore_axis_name="s")

  @functools.partial(
      pl.kernel,
      mesh=mesh,
      out_type=[
          jax.ShapeDtypeStruct((_NC, _C_PAD, _D), jnp.float32),
          jax.ShapeDtypeStruct((_NC, _C_PAD, _CW), jnp.float32),
          jax.ShapeDtypeStruct((_NC, 2 * _NS, _D), jnp.float32),
          jax.ShapeDtypeStruct((_NC, 2 * _NS, _CW), jnp.float32),
      ],
      scratch_types=[
          pltpu.VMEM((_CHUNK, _D), jnp.float32),
          pltpu.VMEM((112,), jnp.int32),
          pltpu.VMEM((_D,), jnp.float32),
          pltpu.VMEM((_CW,), jnp.float32),
          pltpu.VMEM_SHARED((_C_PAD, _D), jnp.float32),
          pltpu.VMEM_SHARED((_C_PAD, _CW), jnp.float32),
          pltpu.VMEM_SHARED((2 * _NS, _D), jnp.float32),
          pltpu.VMEM_SHARED((2 * _NS, _CW), jnp.float32),
      ],
  )
  def k(x_hbm, a_hbm, zrow_hbm, zcnt_hbm,
        out_hbm, cnt_hbm, side_hbm, scnt_hbm,
        rows_v, idx_v, fl_v, flc_v, acc_sh, cnt_sh, side_sh, sidec_sh):
    c = lax.axis_index("c")
    s = lax.axis_index("s")
    wid = c * _NS + s

    pltpu.sync_copy(zrow_hbm, acc_sh.at[pl.ds(s * _SLAB, _SLAB)])
    pltpu.sync_copy(zcnt_hbm, cnt_sh.at[pl.ds(s * _SLAB, _SLAB)])
    zero16 = jnp.zeros((16,), jnp.float32)
    for j in range(_NG):
      fl_v[pl.ds(j * 16, 16)] = zero16
    for j in range(_CW // 16):
      flc_v[pl.ds(j * 16, 16)] = zero16
    pltpu.sync_copy(fl_v, side_sh.at[2 * s])
    pltpu.sync_copy(flc_v, sidec_sh.at[2 * s])

    c0 = _BASE * wid + jnp.minimum(wid, _EXTRA)
    n = _BASE + jnp.where(wid < _EXTRA, 1, 0)

    ones16 = jnp.ones((16,), jnp.float32)
    zero16f = jnp.zeros((16,), jnp.float32)

    def flush(row_off, is_b, accs, cnt_vec, fc):
      @pl.when(jnp.logical_and(is_b, fc >= 1))
      def _():
        for j in range(_NG):
          fl_v[pl.ds(j * 16, 16)] = accs[j]
        for j in range(_CW // 16):
          flc_v[pl.ds(j * 16, 16)] = cnt_vec

        @pl.when(fc == 1)
        def _():
          pltpu.sync_copy(fl_v, side_sh.at[2 * s])
          pltpu.sync_copy(flc_v, sidec_sh.at[2 * s])

        @pl.when(fc >= 2)
        def _():
          row = idx_v[pl.ds(row_off, 16)][0]
          pltpu.sync_copy(fl_v, acc_sh.at[row])
          pltpu.sync_copy(flc_v, cnt_sh.at[row])

    def chunk_body(i, carry):
      start = pl.multiple_of((c0 + i) * _CHUNK, _CHUNK)
      pltpu.sync_copy(a_hbm.at[pl.ds(start, _CHUNK)],
                      idx_v.at[pl.ds(16, _CHUNK)])
      pltpu.sync_copy(x_hbm.at[pl.ds(start, _CHUNK)], rows_v)

      def group_body(g, carry):
        accs, cnt_vec, fc = carry

        for r in range(16):
          row16 = g * 16 + r
          id_prev = idx_v[pl.ds(15 + row16, 16)][0]
          id_cur = idx_v[pl.ds(16 + row16, 16)][0]
          is_b = id_cur != id_prev
          flush(row16 + 15, is_b, accs, cnt_vec, fc)
          keep = jnp.full((16,), 1.0 - is_b.astype(jnp.float32))
          accs = tuple(
              accs[j] * keep + rows_v[row16, pl.ds(j * 16, 16)]
              for j in range(_NG))
          cnt_vec = cnt_vec * keep + ones16
          fc = fc + is_b.astype(jnp.int32)

        return (accs, cnt_vec, fc)

      out = lax.fori_loop(0, _GPC, group_body, carry)
      idx_v[pl.ds(0, 16)] = idx_v[pl.ds(_CHUNK, 16)]
      return out

    idx_v[pl.ds(0, 16)] = jnp.full((16,), -1, jnp.int32)
    init_accs = tuple(jnp.zeros((16,), jnp.float32) for _ in range(_NG))
    init = (init_accs, jnp.zeros((16,), jnp.float32), jnp.int32(0))
    accs, cnt_vec, fc = lax.fori_loop(0, n, chunk_body, init)

    for j in range(_NG):
      fl_v[pl.ds(j * 16, 16)] = accs[j]
    for j in range(_CW // 16):
      flc_v[pl.ds(j * 16, 16)] = cnt_vec
    pltpu.sync_copy(fl_v, side_sh.at[2 * s + 1])
    pltpu.sync_copy(flc_v, sidec_sh.at[2 * s + 1])

    plsc.subcore_barrier()
    pltpu.sync_copy(acc_sh.at[pl.ds(s * _SLAB, _SLAB)],
                    out_hbm.at[c, pl.ds(s * _SLAB, _SLAB)])
    pltpu.sync_copy(cnt_sh.at[pl.ds(s * _SLAB, _SLAB)],
                    cnt_hbm.at[c, pl.ds(s * _SLAB, _SLAB)])

    @pl.when(s == 0)
    def _():
      pltpu.sync_copy(side_sh, side_hbm.at[c])
      pltpu.sync_copy(sidec_sh, scnt_hbm.at[c])

  return k(x, a, zrow, zcnt)


def _combine(main, mcnt, side2, scnt2, sids):
  def body(p_ref, c_ref, s_ref, sc_ref, id_ref, o_ref):
    sums = p_ref[0, : _N_CLUSTERS] + p_ref[1, : _N_CLUSTERS]
    cnt = c_ref[0, : _N_CLUSTERS, 0:1] + c_ref[1, : _N_CLUSTERS, 0:1]
    rows = lax.broadcasted_iota(jnp.int32, (_N_CLUSTERS, 2 * _NW), 0)
    onehot = jnp.where(rows == id_ref[...], 1.0, 0.0)
    sums = sums + jnp.dot(onehot, s_ref[...],
                          preferred_element_type=jnp.float32)
    cnt = cnt + jnp.dot(onehot, sc_ref[...],
                        preferred_element_type=jnp.float32)[:, 0:1]
    o_ref[...] = sums / jnp.maximum(cnt, 1.0)

  return pl.pallas_call(
      body,
      out_shape=jax.ShapeDtypeStruct((_N_CLUSTERS, _D), jnp.float32),
  )(main, mcnt, side2, scnt2, sids)


def kernel(X, assign):
  a = assign.astype(jnp.int32)
  zrow = jnp.zeros((_SLAB, _D), jnp.float32)
  zcnt = jnp.zeros((_SLAB, _CW), jnp.float32)
  main, mcnt, side, scnt = _sc_partial_sums(X, a, zrow, zcnt)
  bounds = [_tile_bounds(w) for w in range(_NW)]
  edge_rows = jnp.array([r for (b, e) in bounds for r in (b, e - 1)],
                        dtype=jnp.int32)
  sids = a[edge_rows].reshape(1, 2 * _NW)
  side2 = side.reshape(2 * _NW, _D)
  scnt2 = scnt.reshape(2 * _NW, _CW)
  return _combine(main, mcnt, side2, scnt2, sids)


# speedup vs baseline: 1.9599x; 1.1289x over previous
import functools

import jax
import jax.numpy as jnp
from jax import lax
from jax.experimental import pallas as pl
from jax.experimental.pallas import tpu as pltpu
from jax.experimental.pallas import tpu_sc as plsc

_N_ROWS = 100000
_D = 512
_NG = _D // 16
_N_CLUSTERS = 1000
_C_PAD = 1024
_NC = 2
_NS = 16
_NW = _NC * _NS
_CHUNK = 80
_GPC = _CHUNK // 16
_N_CHUNKS = _N_ROWS // _CHUNK
_BASE = _N_CHUNKS // _NW
_EXTRA = _N_CHUNKS - _BASE * _NW
_SLAB = _C_PAD // _NS


def _tile_bounds(w):
  c0 = _BASE * w + min(w, _EXTRA)
  n = _BASE + (1 if w < _EXTRA else 0)
  return c0 * _CHUNK, (c0 + n) * _CHUNK


def _sc_partial_sums(x, a, zrow, zcnt):
  mesh = plsc.VectorSubcoreMesh(core_axis_name="c", subcore_axis_name="s")

  @functools.partial(
      pl.kernel,
      mesh=mesh,
      out_type=[
          jax.ShapeDtypeStruct((_NC, _C_PAD, _D), jnp.float32),
          jax.ShapeDtypeStruct((_NC, 16 * _C_PAD), jnp.float32),
          jax.ShapeDtypeStruct((_NC, 2 * _NS, _D), jnp.float32),
          jax.ShapeDtypeStruct((_NC, 32 * _NS), jnp.float32),
      ],
      scratch_types=[
          pltpu.VMEM((_CHUNK, _D), jnp.float32),
          pltpu.VMEM((112,), jnp.int32),
          pltpu.VMEM((_D,), jnp.float32),
          pltpu.VMEM((16,), jnp.float32),
          pltpu.SMEM((1,), jnp.int32),
          pltpu.VMEM_SHARED((_C_PAD, _D), jnp.float32),
          pltpu.VMEM_SHARED((16 * _C_PAD,), jnp.float32),
          pltpu.VMEM_SHARED((2 * _NS, _D), jnp.float32),
          pltpu.VMEM_SHARED((32 * _NS,), jnp.float32),
      ],
  )
  def k(x_hbm, a_hbm, zrow_hbm, zcnt_hbm,
        out_hbm, cnt_hbm, side_hbm, scnt_hbm,
        rows_v, idx_v, acc_v, cntb_v, fc_v, acc_sh, cnt_sh, side_sh, sidec_sh):
    c = lax.axis_index("c")
    s = lax.axis_index("s")
    wid = c * _NS + s

    ones16 = jnp.ones((16,), jnp.float32)
    zero16 = jnp.zeros((16,), jnp.float32)

    pltpu.sync_copy(zrow_hbm, acc_sh.at[pl.ds(s * _SLAB, _SLAB)])
    pltpu.sync_copy(zcnt_hbm, cnt_sh.at[pl.ds(s * 16 * _SLAB, 16 * _SLAB)])
    for j in range(_NG):
      acc_v[pl.ds(j * 16, 16)] = zero16
    cntb_v[pl.ds(0, 16)] = zero16
    pltpu.sync_copy(acc_v, side_sh.at[2 * s])
    pltpu.sync_copy(cntb_v, sidec_sh.at[pl.ds(32 * s, 16)])
    fc_v[0] = 0

    c0 = _BASE * wid + jnp.minimum(wid, _EXTRA)
    n = _BASE + jnp.where(wid < _EXTRA, 1, 0)

    def flush(dest_row):
      fc = fc_v[0]

      @pl.when(fc == 1)
      def _():
        pltpu.sync_copy(acc_v, side_sh.at[2 * s])
        pltpu.sync_copy(cntb_v, sidec_sh.at[pl.ds(32 * s, 16)])

      @pl.when(fc >= 2)
      def _():
        pltpu.sync_copy(acc_v, acc_sh.at[dest_row])
        pltpu.sync_copy(cntb_v, cnt_sh.at[pl.ds(dest_row * 16, 16)])

      fc_v[0] = fc + 1

    def chunk_body(i, carry):
      start = pl.multiple_of((c0 + i) * _CHUNK, _CHUNK)
      pltpu.sync_copy(a_hbm.at[pl.ds(start, _CHUNK)],
                      idx_v.at[pl.ds(16, _CHUNK)])
      pltpu.sync_copy(x_hbm.at[pl.ds(start, _CHUNK)], rows_v)

      def group_body(g, carry):
        g16 = g * 16
        id_before = idx_v[pl.ds(15 + g16, 16)][0]
        id_last = idx_v[pl.ds(31 + g16, 16)][0]

        @pl.when(id_before == id_last)
        def _():
          for j in range(_NG):
            sl = pl.ds(j * 16, 16)
            tot = rows_v[g16, sl]
            for r in range(1, 16):
              tot = tot + rows_v[g16 + r, sl]
            acc_v[sl] = acc_v[sl] + tot
          cntb_v[pl.ds(0, 16)] = cntb_v[pl.ds(0, 16)] + (ones16 * 16.0)

        @pl.when(id_before != id_last)
        def _():
          for r in range(16):
            id_prev = idx_v[pl.ds(15 + g16 + r, 16)][0]
            id_cur = idx_v[pl.ds(16 + g16 + r, 16)][0]

            @pl.when(id_cur != id_prev)
            def _():
              flush(id_prev)
              for j in range(_NG):
                sl = pl.ds(j * 16, 16)
                acc_v[sl] = rows_v[g16 + r, sl]
              cntb_v[pl.ds(0, 16)] = ones16

            @pl.when(id_cur == id_prev)
            def _():
              for j in range(_NG):
                sl = pl.ds(j * 16, 16)
                acc_v[sl] = acc_v[sl] + rows_v[g16 + r, sl]
              cntb_v[pl.ds(0, 16)] = cntb_v[pl.ds(0, 16)] + ones16

        return carry

      out = lax.fori_loop(0, _GPC, group_body, carry)
      idx_v[pl.ds(0, 16)] = idx_v[pl.ds(_CHUNK, 16)]
      return out

    idx_v[pl.ds(0, 16)] = jnp.full((16,), -1, jnp.int32)
    lax.fori_loop(0, n, chunk_body, 0)

    pltpu.sync_copy(acc_v, side_sh.at[2 * s + 1])
    pltpu.sync_copy(cntb_v, sidec_sh.at[pl.ds(32 * s + 16, 16)])

    plsc.subcore_barrier()
    pltpu.sync_copy(acc_sh.at[pl.ds(s * _SLAB, _SLAB)],
                    out_hbm.at[c, pl.ds(s * _SLAB, _SLAB)])
    pltpu.sync_copy(cnt_sh.at[pl.ds(s * 16 * _SLAB, 16 * _SLAB)],
                    cnt_hbm.at[c, pl.ds(s * 16 * _SLAB, 16 * _SLAB)])

    @pl.when(s == 0)
    def _():
      pltpu.sync_copy(side_sh, side_hbm.at[c])
      pltpu.sync_copy(sidec_sh, scnt_hbm.at[c])

  return k(x, a, zrow, zcnt)


def _combine(main, cnt_col, side2, scnt2, sids):
  def body(p_ref, c_ref, s_ref, sc_ref, id_ref, o_ref):
    sums = p_ref[0, : _N_CLUSTERS] + p_ref[1, : _N_CLUSTERS]
    cnt = c_ref[: _N_CLUSTERS]
    rows = lax.broadcasted_iota(jnp.int32, (_N_CLUSTERS, 2 * _NW), 0)
    ids = jnp.broadcast_to(id_ref[...], (_N_CLUSTERS, 2 * _NW))
    onehot = jnp.where(rows == ids, 1.0, 0.0)
    sums = sums + jnp.dot(onehot, s_ref[...],
                          preferred_element_type=jnp.float32)
    cnt = cnt + jnp.dot(onehot, sc_ref[...],
                        preferred_element_type=jnp.float32)[:, 0:1]
    o_ref[...] = sums / jnp.maximum(cnt, 1.0)

  return pl.pallas_call(
      body,
      out_shape=jax.ShapeDtypeStruct((_N_CLUSTERS, _D), jnp.float32),
  )(main, cnt_col, side2, scnt2, sids)


def kernel(X, assign):
  a = assign.astype(jnp.int32)
  zrow = jnp.zeros((_SLAB, _D), jnp.float32)
  zcnt = jnp.zeros((16 * _SLAB,), jnp.float32)
  main, mcnt, side, scnt = _sc_partial_sums(X, a, zrow, zcnt)
  bounds = [_tile_bounds(w) for w in range(_NW)]
  edge_rows = jnp.array([r for (b, e) in bounds for r in (b, e - 1)],
                        dtype=jnp.int32)
  sids = a[edge_rows].reshape(1, 2 * _NW)
  cnt_col = (mcnt[0] + mcnt[1]).reshape(_C_PAD, 16)[:, :1]
  side2 = side.reshape(2 * _NW, _D)
  scnt2 = jnp.broadcast_to(
      scnt.reshape(_NC, _NS, 2, 16)[:, :, :, 0].reshape(2 * _NW, 1),
      (2 * _NW, 16))
  return _combine(main, cnt_col, side2, scnt2, sids)


# speedup vs baseline: 2.5142x; 1.2828x over previous
import functools

import jax
import jax.numpy as jnp
from jax import lax
from jax.experimental import pallas as pl
from jax.experimental.pallas import tpu as pltpu
from jax.experimental.pallas import tpu_sc as plsc

_N_ROWS = 100000
_D = 512
_NG = _D // 16
_N_CLUSTERS = 1000
_C_PAD = 1024
_NC = 2
_NS = 16
_NW = _NC * _NS
_CHUNK = 80
_GPC = _CHUNK // 16
_N_CHUNKS = _N_ROWS // _CHUNK
_BASE = _N_CHUNKS // _NW
_EXTRA = _N_CHUNKS - _BASE * _NW
_SLAB = _C_PAD // _NS


def _tile_bounds(w):
  c0 = _BASE * w + min(w, _EXTRA)
  n = _BASE + (1 if w < _EXTRA else 0)
  return c0 * _CHUNK, (c0 + n) * _CHUNK


def _sc_partial_sums(x, a, zrow, zcnt):
  mesh = plsc.VectorSubcoreMesh(core_axis_name="c", subcore_axis_name="s")

  @functools.partial(
      pl.kernel,
      mesh=mesh,
      out_type=[
          jax.ShapeDtypeStruct((_NC, _C_PAD, _D), jnp.float32),
          jax.ShapeDtypeStruct((_NC, 16 * _C_PAD), jnp.float32),
          jax.ShapeDtypeStruct((_NC, 2 * _NS, _D), jnp.float32),
          jax.ShapeDtypeStruct((_NC, 32 * _NS), jnp.float32),
      ],
      scratch_types=[
          pltpu.VMEM((2, _CHUNK, _D), jnp.float32),
          pltpu.VMEM((2, 112), jnp.int32),
          pltpu.SemaphoreType.DMA,
          pltpu.SemaphoreType.DMA,
          pltpu.VMEM((_D,), jnp.float32),
          pltpu.VMEM((16,), jnp.float32),
          pltpu.SMEM((1,), jnp.int32),
          pltpu.VMEM_SHARED((_C_PAD, _D), jnp.float32),
          pltpu.VMEM_SHARED((16 * _C_PAD,), jnp.float32),
          pltpu.VMEM_SHARED((2 * _NS, _D), jnp.float32),
          pltpu.VMEM_SHARED((32 * _NS,), jnp.float32),
      ],
  )
  def k(x_hbm, a_hbm, zrow_hbm, zcnt_hbm,
        out_hbm, cnt_hbm, side_hbm, scnt_hbm,
        rows_v, idx_v, rsem, isem, acc_v, cntb_v, fc_v,
        acc_sh, cnt_sh, side_sh, sidec_sh):
    c = lax.axis_index("c")
    s = lax.axis_index("s")
    wid = c * _NS + s

    ones16 = jnp.ones((16,), jnp.float32)
    zero16 = jnp.zeros((16,), jnp.float32)

    pltpu.sync_copy(zrow_hbm, acc_sh.at[pl.ds(s * _SLAB, _SLAB)])
    pltpu.sync_copy(zcnt_hbm, cnt_sh.at[pl.ds(s * 16 * _SLAB, 16 * _SLAB)])
    for j in range(_NG):
      acc_v[pl.ds(j * 16, 16)] = zero16
    cntb_v[pl.ds(0, 16)] = zero16
    pltpu.sync_copy(acc_v, side_sh.at[2 * s])
    pltpu.sync_copy(cntb_v, sidec_sh.at[pl.ds(32 * s, 16)])
    fc_v[0] = 0

    c0 = _BASE * wid + jnp.minimum(wid, _EXTRA)
    n = _BASE + jnp.where(wid < _EXTRA, 1, 0)

    def flush(dest_row):
      fc = fc_v[0]

      @pl.when(fc == 1)
      def _():
        pltpu.sync_copy(acc_v, side_sh.at[2 * s])
        pltpu.sync_copy(cntb_v, sidec_sh.at[pl.ds(32 * s, 16)])

      @pl.when(fc >= 2)
      def _():
        pltpu.sync_copy(acc_v, acc_sh.at[dest_row])
        pltpu.sync_copy(cntb_v, cnt_sh.at[pl.ds(dest_row * 16, 16)])

      fc_v[0] = fc + 1

    def chunk_refs(ci, b):
      start = pl.multiple_of((c0 + ci) * _CHUNK, _CHUNK)
      return ((a_hbm.at[pl.ds(start, _CHUNK)], idx_v.at[b, pl.ds(16, _CHUNK)]),
              (x_hbm.at[pl.ds(start, _CHUNK)], rows_v.at[b]))

    def start_chunk(ci, b):
      (asrc, adst), (xsrc, xdst) = chunk_refs(ci, b)
      pltpu.async_copy(asrc, adst, isem)
      pltpu.async_copy(xsrc, xdst, rsem)

    def wait_chunk(ci, b):
      (asrc, adst), (xsrc, xdst) = chunk_refs(ci, b)
      pltpu.make_async_copy(asrc, adst, isem).wait()
      pltpu.make_async_copy(xsrc, xdst, rsem).wait()

    def chunk_body(i, carry):
      b = lax.rem(i, 2)
      wait_chunk(i, b)
      idx_v[1 - b, pl.ds(0, 16)] = idx_v[b, pl.ds(_CHUNK, 16)]

      @pl.when(i + 1 < n)
      def _():
        start_chunk(i + 1, 1 - b)

      def group_body(g, carry):
        g16 = g * 16
        id_before = idx_v[b, pl.ds(15 + g16, 16)][0]
        id_last = idx_v[b, pl.ds(31 + g16, 16)][0]

        @pl.when(id_before == id_last)
        def _():
          for j in range(_NG):
            sl = pl.ds(j * 16, 16)
            vals = [rows_v[b, g16 + r, sl] for r in range(16)]
            while len(vals) > 1:
              vals = [vals[t] + vals[t + 1] for t in range(0, len(vals), 2)]
            acc_v[sl] = acc_v[sl] + vals[0]
          cntb_v[pl.ds(0, 16)] = cntb_v[pl.ds(0, 16)] + (ones16 * 16.0)

        @pl.when(id_before != id_last)
        def _():
          for r in range(16):
            id_prev = idx_v[b, pl.ds(15 + g16 + r, 16)][0]
            id_cur = idx_v[b, pl.ds(16 + g16 + r, 16)][0]

            @pl.when(id_cur != id_prev)
            def _():
              flush(id_prev)
              for j in range(_NG):
                sl = pl.ds(j * 16, 16)
                acc_v[sl] = rows_v[b, g16 + r, sl]
              cntb_v[pl.ds(0, 16)] = ones16

            @pl.when(id_cur == id_prev)
            def _():
              for j in range(_NG):
                sl = pl.ds(j * 16, 16)
                acc_v[sl] = acc_v[sl] + rows_v[b, g16 + r, sl]
              cntb_v[pl.ds(0, 16)] = cntb_v[pl.ds(0, 16)] + ones16

        return carry

      return lax.fori_loop(0, _GPC, group_body, carry)

    idx_v[0, pl.ds(0, 16)] = jnp.full((16,), -1, jnp.int32)
    start_chunk(0, 0)
    lax.fori_loop(0, n, chunk_body, 0)

    pltpu.sync_copy(acc_v, side_sh.at[2 * s + 1])
    pltpu.sync_copy(cntb_v, sidec_sh.at[pl.ds(32 * s + 16, 16)])

    plsc.subcore_barrier()
    pltpu.sync_copy(acc_sh.at[pl.ds(s * _SLAB, _SLAB)],
                    out_hbm.at[c, pl.ds(s * _SLAB, _SLAB)])
    pltpu.sync_copy(cnt_sh.at[pl.ds(s * 16 * _SLAB, 16 * _SLAB)],
                    cnt_hbm.at[c, pl.ds(s * 16 * _SLAB, 16 * _SLAB)])

    @pl.when(s == 0)
    def _():
      pltpu.sync_copy(side_sh, side_hbm.at[c])
      pltpu.sync_copy(sidec_sh, scnt_hbm.at[c])

  return k(x, a, zrow, zcnt)


def _combine(main, cnt_col, side2, scnt2, sids):
  def body(p_ref, c_ref, s_ref, sc_ref, id_ref, o_ref):
    sums = p_ref[0, : _N_CLUSTERS] + p_ref[1, : _N_CLUSTERS]
    cnt = c_ref[: _N_CLUSTERS]
    rows = lax.broadcasted_iota(jnp.int32, (_N_CLUSTERS, 2 * _NW), 0)
    ids = jnp.broadcast_to(id_ref[...], (_N_CLUSTERS, 2 * _NW))
    onehot = jnp.where(rows == ids, 1.0, 0.0)
    sums = sums + jnp.dot(onehot, s_ref[...],
                          preferred_element_type=jnp.float32)
    cnt = cnt + jnp.dot(onehot, sc_ref[...],
                        preferred_element_type=jnp.float32)[:, 0:1]
    o_ref[...] = sums / jnp.maximum(cnt, 1.0)

  return pl.pallas_call(
      body,
      out_shape=jax.ShapeDtypeStruct((_N_CLUSTERS, _D), jnp.float32),
  )(main, cnt_col, side2, scnt2, sids)


def kernel(X, assign):
  a = assign.astype(jnp.int32)
  zrow = jnp.zeros((_SLAB, _D), jnp.float32)
  zcnt = jnp.zeros((16 * _SLAB,), jnp.float32)
  main, mcnt, side, scnt = _sc_partial_sums(X, a, zrow, zcnt)
  bounds = [_tile_bounds(w) for w in range(_NW)]
  edge_rows = jnp.array([r for (b, e) in bounds for r in (b, e - 1)],
                        dtype=jnp.int32)
  sids = a[edge_rows].reshape(1, 2 * _NW)
  cnt_col = (mcnt[0] + mcnt[1]).reshape(_C_PAD, 16)[:, :1]
  side2 = side.reshape(2 * _NW, _D)
  scnt2 = jnp.broadcast_to(
      scnt.reshape(_NC, _NS, 2, 16)[:, :, :, 0].reshape(2 * _NW, 1),
      (2 * _NW, 16))
  return _combine(main, cnt_col, side2, scnt2, sids)
